# baseline (speedup 1.0000x reference)
import functools
import jax
import jax.numpy as jnp
from jax import lax
from jax.experimental import pallas as pl
from jax.experimental.pallas import tpu as pltpu
from jax.experimental.pallas import tpu_sc as plsc

NC = 2
NS = 16
L = 16
NW = NC * NS
BATCH = 16384
LAT = 64
BPW = BATCH // NW
GROUPS = BPW // L

_mesh = plsc.VectorSubcoreMesh(
    core_axis_name="c", subcore_axis_name="s", num_cores=NC, num_subcores=NS
)


@functools.partial(
    pl.kernel,
    out_type=jax.ShapeDtypeStruct((BATCH,), jnp.float32),
    mesh=_mesh,
    scratch_types=[
        pltpu.VMEM((BPW,), jnp.int32),
        pltpu.VMEM((BPW,), jnp.int32),
        pltpu.VMEM((BPW, LAT), jnp.float32),
        pltpu.VMEM((BPW, LAT), jnp.float32),
        pltpu.VMEM((BPW,), jnp.float32),
        pltpu.VMEM((L,), jnp.float32),
        pltpu.VMEM((L,), jnp.float32),
        pltpu.SemaphoreType.DMA,
        pltpu.SemaphoreType.DMA,
    ],
    compiler_params=pltpu.CompilerParams(
        needs_layout_passes=False, use_tc_tiling_on_sc=False
    ),
)
def _gmf_sc(users_hbm, items_hbm, utab_hbm, itab_hbm, w_hbm, b_hbm,
            out_hbm, uidx_v, iidx_v, urows_v, irows_v, out_v, w_v, b_v,
            sem_u, sem_i):
    wid = lax.axis_index("s") * NC + lax.axis_index("c")
    base = wid * BPW

    pltpu.sync_copy(users_hbm.at[pl.ds(base, BPW)], uidx_v)
    pltpu.sync_copy(items_hbm.at[pl.ds(base, BPW)], iidx_v)
    pltpu.sync_copy(w_hbm, w_v)
    pltpu.sync_copy(b_hbm, b_v)

    cp_u = pltpu.async_copy(utab_hbm.at[uidx_v], urows_v, sem_u)
    cp_i = pltpu.async_copy(itab_hbm.at[iidx_v], irows_v, sem_i)
    cp_u.wait()
    cp_i.wait()

    w = w_v[...]
    b = b_v[...]
    lane = lax.iota(jnp.int32, L)

    def group_body(g, carry):
        row = lane + g * L
        acc = jnp.zeros((L,), jnp.float32)
        for c in range(LAT):
            col = jnp.full((L,), c, jnp.int32)
            u = plsc.load_gather(urows_v, [row, col])
            v = plsc.load_gather(irows_v, [row, col])
            acc = acc + u * v
        z = acc * w + b
        out_v[pl.ds(g * L, L)] = 1.0 / (1.0 + jnp.exp(-z))
        return carry

    lax.fori_loop(0, GROUPS, group_body, 0)
    pltpu.sync_copy(out_v, out_hbm.at[pl.ds(base, BPW)])


def kernel(users, items, user_table, item_table, W, b):
    w16 = jnp.broadcast_to(W.reshape(()), (L,)).astype(jnp.float32)
    b16 = jnp.broadcast_to(b.reshape(()), (L,)).astype(jnp.float32)
    out = _gmf_sc(users, items, user_table, item_table, w16, b16)
    return out[:, None]


# speedup vs baseline: 2.1639x; 2.1639x over previous
import functools
import jax
import jax.numpy as jnp
from jax import lax
from jax.experimental import pallas as pl
from jax.experimental.pallas import tpu as pltpu
from jax.experimental.pallas import tpu_sc as plsc

NC = 2
NS = 16
L = 16
NW = NC * NS
BATCH = 16384
LAT = 64
BPW = BATCH // NW
GROUPS = BPW // L
TB = 128

_mesh = plsc.VectorSubcoreMesh(
    core_axis_name="c", subcore_axis_name="s", num_cores=NC, num_subcores=NS
)


@functools.partial(
    pl.kernel,
    out_type=jax.ShapeDtypeStruct((BATCH,), jnp.float32),
    mesh=_mesh,
    scratch_types=[
        pltpu.VMEM((BPW,), jnp.int32),
        pltpu.VMEM((BPW,), jnp.int32),
        pltpu.VMEM((2, LAT, TB), jnp.float32),
        pltpu.VMEM((2, LAT, TB), jnp.float32),
        pltpu.VMEM((L * LAT,), jnp.float32),
        pltpu.VMEM((L * LAT,), jnp.float32),
        pltpu.VMEM((BPW,), jnp.float32),
        pltpu.VMEM((L,), jnp.float32),
        pltpu.VMEM((L,), jnp.float32),
        pltpu.SemaphoreType.DMA,
        pltpu.SemaphoreType.DMA,
        pltpu.SemaphoreType.DMA,
        pltpu.SemaphoreType.DMA,
    ],
    compiler_params=pltpu.CompilerParams(needs_layout_passes=False),
)
def _gmf_sc(users_hbm, items_hbm, utab_hbm, itab_hbm, w_hbm, b_hbm,
            out_hbm, uidx_v, iidx_v, uring, iring,
            ugrp, igrp, out_v, w_v, b_v, sem_u0, sem_u1, sem_i0, sem_i1):
    wid = lax.axis_index("s") * NC + lax.axis_index("c")
    base = wid * BPW

    pltpu.sync_copy(users_hbm.at[pl.ds(base, BPW)], uidx_v)
    pltpu.sync_copy(items_hbm.at[pl.ds(base, BPW)], iidx_v)
    pltpu.sync_copy(w_hbm, w_v)
    pltpu.sync_copy(b_hbm, b_v)

    w = w_v[...]
    b = b_v[...]
    lane = lax.iota(jnp.int32, L)
    sems_u = (sem_u0, sem_u1)
    sems_i = (sem_i0, sem_i1)

    def fire(su, si, ph):
        u0 = pl.multiple_of((su // TB) * TB, TB)
        i0 = pl.multiple_of((si // TB) * TB, TB)
        pltpu.async_copy(
            utab_hbm.at[:, pl.ds(u0, TB)], uring.at[ph], sems_u[ph])
        pltpu.async_copy(
            itab_hbm.at[:, pl.ds(i0, TB)], iring.at[ph], sems_i[ph])

    def drain(ph):
        pltpu.make_async_copy(
            utab_hbm.at[:, pl.ds(0, TB)], uring.at[ph], sems_u[ph]).wait()
        pltpu.make_async_copy(
            itab_hbm.at[:, pl.ds(0, TB)], iring.at[ph], sems_i[ph]).wait()

    def extract(su, si, j, ph):
        urem = jnp.full((L,), su % TB, jnp.int32)
        irem = jnp.full((L,), si % TB, jnp.int32)
        pvec = jnp.full((L,), ph, jnp.int32)
        for k in range(LAT // L):
            cvec = lane + k * L
            ugrp[pl.ds(j * LAT + k * L, L)] = plsc.load_gather(
                uring, [pvec, cvec, urem])
            igrp[pl.ds(j * LAT + k * L, L)] = plsc.load_gather(
                iring, [pvec, cvec, irem])

    uvec0 = uidx_v[pl.ds(0, L)]
    ivec0 = iidx_v[pl.ds(0, L)]
    fire(uvec0[0], ivec0[0], 0)

    def group_body(g, carry):
        uvec = uidx_v[pl.ds(g * L, L)]
        ivec = iidx_v[pl.ds(g * L, L)]
        for j in range(L):
            ph = j % 2
            if j < L - 1:
                fire(uvec[j + 1], ivec[j + 1], (j + 1) % 2)
            else:
                @pl.when(g + 1 < GROUPS)
                def _():
                    unext = uidx_v[pl.ds((g + 1) * L, L)]
                    inext = iidx_v[pl.ds((g + 1) * L, L)]
                    fire(unext[0], inext[0], 0)

            drain(ph)
            extract(uvec[j], ivec[j], j, ph)

        acc = jnp.zeros((L,), jnp.float32)
        flat0 = lane * LAT
        for c in range(LAT):
            u = plsc.load_gather(ugrp, [flat0 + c])
            v = plsc.load_gather(igrp, [flat0 + c])
            acc = acc + u * v
        z = acc * w + b
        out_v[pl.ds(g * L, L)] = 1.0 / (1.0 + jnp.exp(-z))
        return carry

    lax.fori_loop(0, GROUPS, group_body, 0)
    pltpu.sync_copy(out_v, out_hbm.at[pl.ds(base, BPW)])


def kernel(users, items, user_table, item_table, W, b):
    w16 = jnp.broadcast_to(W.reshape(()), (L,)).astype(jnp.float32)
    b16 = jnp.broadcast_to(b.reshape(()), (L,)).astype(jnp.float32)
    out = _gmf_sc(users, items, user_table.T, item_table.T, w16, b16)
    return out[:, None]


# speedup vs baseline: 2.6319x; 1.2163x over previous
import functools
import jax
import jax.numpy as jnp
from jax import lax
from jax.experimental import pallas as pl
from jax.experimental.pallas import tpu as pltpu
from jax.experimental.pallas import tpu_sc as plsc

NC = 2
NS = 16
L = 16
NW = NC * NS
BATCH = 16384
LAT = 64
BPW = BATCH // NW
GROUPS = BPW // L
TB = 128

_mesh = plsc.VectorSubcoreMesh(
    core_axis_name="c", subcore_axis_name="s", num_cores=NC, num_subcores=NS
)


@functools.partial(
    pl.kernel,
    out_type=jax.ShapeDtypeStruct((BATCH,), jnp.float32),
    mesh=_mesh,
    scratch_types=[
        pltpu.VMEM((BPW,), jnp.int32),
        pltpu.VMEM((BPW,), jnp.int32),
        pltpu.VMEM((4, LAT, TB), jnp.float32),
        pltpu.VMEM((4, LAT, TB), jnp.float32),
        pltpu.VMEM((L * LAT,), jnp.float32),
        pltpu.VMEM((L * LAT,), jnp.float32),
        pltpu.VMEM((BPW,), jnp.float32),
        pltpu.VMEM((L,), jnp.float32),
        pltpu.VMEM((L,), jnp.float32),
        pltpu.SemaphoreType.DMA,
        pltpu.SemaphoreType.DMA,
        pltpu.SemaphoreType.DMA,
        pltpu.SemaphoreType.DMA,
        pltpu.SemaphoreType.DMA,
        pltpu.SemaphoreType.DMA,
        pltpu.SemaphoreType.DMA,
        pltpu.SemaphoreType.DMA,
    ],
    compiler_params=pltpu.CompilerParams(needs_layout_passes=False),
)
def _gmf_sc(users_hbm, items_hbm, utab_hbm, itab_hbm, w_hbm, b_hbm,
            out_hbm, uidx_v, iidx_v, uring, iring,
            ugrp, igrp, out_v, w_v, b_v,
            sem_u0, sem_u1, sem_u2, sem_u3, sem_i0, sem_i1, sem_i2, sem_i3):
    wid = lax.axis_index("s") * NC + lax.axis_index("c")
    base = wid * BPW

    pltpu.sync_copy(users_hbm.at[pl.ds(base, BPW)], uidx_v)
    pltpu.sync_copy(items_hbm.at[pl.ds(base, BPW)], iidx_v)
    pltpu.sync_copy(w_hbm, w_v)
    pltpu.sync_copy(b_hbm, b_v)

    w = w_v[...]
    b = b_v[...]
    lane = lax.iota(jnp.int32, L)
    sems_u = (sem_u0, sem_u1, sem_u2, sem_u3)
    sems_i = (sem_i0, sem_i1, sem_i2, sem_i3)

    def fire(su, si, ph):
        u0 = pl.multiple_of((su // TB) * TB, TB)
        i0 = pl.multiple_of((si // TB) * TB, TB)
        pltpu.async_copy(
            utab_hbm.at[:, pl.ds(u0, TB)], uring.at[ph], sems_u[ph])
        pltpu.async_copy(
            itab_hbm.at[:, pl.ds(i0, TB)], iring.at[ph], sems_i[ph])

    def drain(ph):
        pltpu.make_async_copy(
            utab_hbm.at[:, pl.ds(0, TB)], uring.at[ph], sems_u[ph]).wait()
        pltpu.make_async_copy(
            itab_hbm.at[:, pl.ds(0, TB)], iring.at[ph], sems_i[ph]).wait()

    def extract(su, si, j, ph):
        urem = jnp.full((L,), su % TB, jnp.int32)
        irem = jnp.full((L,), si % TB, jnp.int32)
        pvec = jnp.full((L,), ph, jnp.int32)
        for k in range(LAT // L):
            cvec = lane + k * L
            ugrp[pl.ds(j * LAT + k * L, L)] = plsc.load_gather(
                uring, [pvec, cvec, urem])
            igrp[pl.ds(j * LAT + k * L, L)] = plsc.load_gather(
                iring, [pvec, cvec, irem])

    PH = 4
    uvec0 = uidx_v[pl.ds(0, L)]
    ivec0 = iidx_v[pl.ds(0, L)]
    for p in range(PH - 1):
        fire(uvec0[p], ivec0[p], p)

    def group_body(g, carry):
        uvec = uidx_v[pl.ds(g * L, L)]
        ivec = iidx_v[pl.ds(g * L, L)]
        for j in range(L):
            ph = j % PH
            ahead = j + PH - 1
            if ahead < L:
                fire(uvec[ahead], ivec[ahead], ahead % PH)
            else:
                @pl.when(g + 1 < GROUPS)
                def _():
                    unext = uidx_v[pl.ds((g + 1) * L, L)]
                    inext = iidx_v[pl.ds((g + 1) * L, L)]
                    fire(unext[ahead - L], inext[ahead - L], ahead % PH)

            drain(ph)
            extract(uvec[j], ivec[j], j, ph)

        acc = jnp.zeros((L,), jnp.float32)
        flat0 = lane * LAT
        for c in range(LAT):
            u = plsc.load_gather(ugrp, [flat0 + c])
            v = plsc.load_gather(igrp, [flat0 + c])
            acc = acc + u * v
        z = acc * w + b
        out_v[pl.ds(g * L, L)] = 1.0 / (1.0 + jnp.exp(-z))
        return carry

    lax.fori_loop(0, GROUPS, group_body, 0)
    pltpu.sync_copy(out_v, out_hbm.at[pl.ds(base, BPW)])


def kernel(users, items, user_table, item_table, W, b):
    w16 = jnp.broadcast_to(W.reshape(()), (L,)).astype(jnp.float32)
    b16 = jnp.broadcast_to(b.reshape(()), (L,)).astype(jnp.float32)
    out = _gmf_sc(users, items, user_table.T, item_table.T, w16, b16)
    return out[:, None]
